# baseline (speedup 1.0000x reference)
import functools

import jax
import jax.numpy as jnp
from jax import lax
from jax.experimental import pallas as pl
from jax.experimental.pallas import tpu as pltpu
from jax.experimental.pallas import tpu_sc as plsc

HIDDEN = 128
SEQ = 200
EPS = 1e-12

NC = 2
NS = 16
NW = NC * NS

CHUNK = 80
NBUF = 2
POS_BUF = 240
LANES = 16
NVEC = HIDDEN // LANES
UNROLL = 2


def _rsqrt(v):
    i = plsc.bitcast(v, jnp.int32)
    i = jnp.int32(0x5F3759DF) - lax.shift_right_logical(i, jnp.int32(1))
    y = plsc.bitcast(i, jnp.float32)
    for _ in range(2):
        y = y * (1.5 - 0.5 * v * y * y)
    return y


def kernel(input_ids, word_emb, pos_emb, gamma, beta):
    B, L = input_ids.shape
    N = B * L
    rows_per_w = N // NW
    n_chunks = rows_per_w // CHUNK
    ids = input_ids.reshape(NW, n_chunks, CHUNK).astype(jnp.int32)
    ids = jnp.pad(ids, ((0, 0), (0, 1), (0, 0)))

    mesh = plsc.VectorSubcoreMesh(core_axis_name="c", subcore_axis_name="s")

    @functools.partial(
        pl.kernel,
        mesh=mesh,
        out_type=jax.ShapeDtypeStruct((N, HIDDEN), jnp.float32),
        compiler_params=pltpu.CompilerParams(needs_layout_passes=False),
        scratch_types=[
            pltpu.VMEM((n_chunks + 1, CHUNK), jnp.int32),
            pltpu.VMEM((NBUF, CHUNK, HIDDEN), jnp.float32),
            pltpu.VMEM((POS_BUF, HIDDEN), jnp.float32),
            pltpu.VMEM((HIDDEN,), jnp.float32),
            pltpu.VMEM((HIDDEN,), jnp.float32),
            [pltpu.SemaphoreType.DMA] * NBUF,
            [pltpu.SemaphoreType.DMA] * NBUF,
        ],
    )
    def emb_kernel(ids_hbm, wemb_hbm, pemb_hbm, gamma_hbm, beta_hbm, out_hbm,
                   idx_all, rows_b, pos_v, g_v, b_v, gsems, wsems):
        wid = lax.axis_index("s") * NC + lax.axis_index("c")

        pltpu.sync_copy(ids_hbm.at[wid], idx_all)
        pltpu.sync_copy(pemb_hbm.at[pl.ds(0, SEQ)], pos_v.at[pl.ds(0, SEQ)])
        pltpu.sync_copy(pemb_hbm.at[pl.ds(0, POS_BUF - SEQ)],
                        pos_v.at[pl.ds(SEQ, POS_BUF - SEQ)])
        pltpu.sync_copy(gamma_hbm, g_v)
        pltpu.sync_copy(beta_hbm, b_v)

        lane = lax.iota(jnp.int32, LANES)
        perms = [lane ^ k for k in (1, 2, 4, 8)]
        gs = [g_v[pl.ds(LANES * j, LANES)] for j in range(NVEC)]
        bs = [b_v[pl.ds(LANES * j, LANES)] for j in range(NVEC)]

        def gather_desc(c, b):
            return pltpu.make_async_copy(
                wemb_hbm.at[idx_all.at[c]], rows_b.at[b], gsems[b])

        def wb_desc(c, b):
            base = wid * rows_per_w + c * CHUNK
            return pltpu.make_async_copy(
                rows_b.at[b], out_hbm.at[pl.ds(base, CHUNK)], wsems[b])

        def compute_chunk(c, b):
            off = lax.rem(c * CHUNK, SEQ)

            @plsc.parallel_loop(0, CHUNK, unroll=UNROLL)
            def row_body(i):
                x = [rows_b[b, i, pl.ds(LANES * j, LANES)] +
                     pos_v[off + i, pl.ds(LANES * j, LANES)]
                     for j in range(NVEC)]
                s = ((x[0] + x[1]) + (x[2] + x[3])) + \
                    ((x[4] + x[5]) + (x[6] + x[7]))
                q = ((x[0] * x[0] + x[1] * x[1]) + (x[2] * x[2] + x[3] * x[3])) + \
                    ((x[4] * x[4] + x[5] * x[5]) + (x[6] * x[6] + x[7] * x[7]))
                for p in perms:
                    s = s + s.at[p].get(mode="promise_in_bounds")
                    q = q + q.at[p].get(mode="promise_in_bounds")
                mean_v = s * (1.0 / HIDDEN)
                var_v = q * (1.0 / HIDDEN) - mean_v * mean_v
                rstd_v = _rsqrt(var_v + EPS)
                for j in range(NVEC):
                    rows_b[b, i, pl.ds(LANES * j, LANES)] = \
                        (x[j] - mean_v) * rstd_v * gs[j] + bs[j]

        gather_desc(0, 0).start()
        gather_desc(1, 1).start()
        gather_desc(0, 0).wait()
        compute_chunk(0, 0)
        wb_desc(0, 0).start()

        pltpu.make_async_copy(
            rows_b.at[0], out_hbm.at[pl.ds(0, CHUNK)], wsems[0]).wait()
        gather_desc(2, 0).start()
        gather_desc(1, 1).wait()
        compute_chunk(1, 1)
        wb_desc(1, 1).start()

        def pair_body(cp, carry):
            for b in range(NBUF):
                c = cp * NBUF + b
                nb = 1 - b
                pltpu.make_async_copy(
                    rows_b.at[nb], out_hbm.at[pl.ds(0, CHUNK)], wsems[nb]
                ).wait()
                gather_desc(c + 1, nb).start()
                gather_desc(c, b).wait()
                compute_chunk(c, b)
                wb_desc(c, b).start()
            return carry

        lax.fori_loop(1, n_chunks // NBUF, pair_body, 0)

        gather_desc(n_chunks, 0).wait()
        wb_desc(n_chunks - 1, 1).wait()

    out = emb_kernel(ids, word_emb, pos_emb, gamma, beta)
    return out.reshape(B, L, HIDDEN)


# speedup vs baseline: 1.1572x; 1.1572x over previous
import functools

import jax
import jax.numpy as jnp
from jax import lax
from jax.experimental import pallas as pl
from jax.experimental.pallas import tpu as pltpu
from jax.experimental.pallas import tpu_sc as plsc

HIDDEN = 128
SEQ = 200
EPS = 1e-12

NC = 2
NS = 16
NW = NC * NS

CHUNK = 40
NBUF = 2
POS_BUF = 200
LANES = 16
NVEC = HIDDEN // LANES
UNROLL = 2


def _rsqrt(v):
    i = plsc.bitcast(v, jnp.int32)
    i = jnp.int32(0x5F3759DF) - lax.shift_right_logical(i, jnp.int32(1))
    y = plsc.bitcast(i, jnp.float32)
    for _ in range(2):
        y = y * (1.5 - 0.5 * v * y * y)
    return y


def kernel(input_ids, word_emb, pos_emb, gamma, beta):
    B, L = input_ids.shape
    N = B * L
    rows_per_w = N // NW
    n_chunks = rows_per_w // CHUNK
    ids = input_ids.reshape(NW, n_chunks, CHUNK).astype(jnp.int32)
    ids = jnp.pad(ids, ((0, 0), (0, 1), (0, 0)))

    mesh = plsc.VectorSubcoreMesh(core_axis_name="c", subcore_axis_name="s")

    @functools.partial(
        pl.kernel,
        mesh=mesh,
        out_type=jax.ShapeDtypeStruct((N, HIDDEN), jnp.float32),
        compiler_params=pltpu.CompilerParams(needs_layout_passes=False),
        scratch_types=[
            pltpu.VMEM((n_chunks + 1, CHUNK), jnp.int32),
            pltpu.VMEM((NBUF, CHUNK, HIDDEN), jnp.float32),
            pltpu.VMEM((POS_BUF, HIDDEN), jnp.float32),
            pltpu.VMEM((HIDDEN,), jnp.float32),
            pltpu.VMEM((HIDDEN,), jnp.float32),
            [pltpu.SemaphoreType.DMA] * NBUF,
            [pltpu.SemaphoreType.DMA] * NBUF,
        ],
    )
    def emb_kernel(ids_hbm, wemb_hbm, pemb_hbm, gamma_hbm, beta_hbm, out_hbm,
                   idx_all, rows_b, pos_v, g_v, b_v, gsems, wsems):
        wid = lax.axis_index("s") * NC + lax.axis_index("c")

        pltpu.sync_copy(ids_hbm.at[wid], idx_all)
        pltpu.sync_copy(pemb_hbm.at[pl.ds(0, SEQ)], pos_v.at[pl.ds(0, SEQ)])
        if POS_BUF > SEQ:
            pltpu.sync_copy(pemb_hbm.at[pl.ds(0, POS_BUF - SEQ)],
                            pos_v.at[pl.ds(SEQ, POS_BUF - SEQ)])
        pltpu.sync_copy(gamma_hbm, g_v)
        pltpu.sync_copy(beta_hbm, b_v)

        lane = lax.iota(jnp.int32, LANES)
        perms = [lane ^ k for k in (1, 2, 4, 8)]
        gs = [g_v[pl.ds(LANES * j, LANES)] for j in range(NVEC)]
        bs = [b_v[pl.ds(LANES * j, LANES)] for j in range(NVEC)]

        def gather_desc(c, b):
            return pltpu.make_async_copy(
                wemb_hbm.at[idx_all.at[c]], rows_b.at[b], gsems[b])

        def wb_desc(c, b):
            base = wid * rows_per_w + c * CHUNK
            return pltpu.make_async_copy(
                rows_b.at[b], out_hbm.at[pl.ds(base, CHUNK)], wsems[b])

        def compute_chunk(c, b):
            off = lax.rem(c * CHUNK, SEQ)

            @plsc.parallel_loop(0, CHUNK, unroll=UNROLL)
            def row_body(i):
                x = [rows_b[b, i, pl.ds(LANES * j, LANES)] +
                     pos_v[off + i, pl.ds(LANES * j, LANES)]
                     for j in range(NVEC)]
                s = ((x[0] + x[1]) + (x[2] + x[3])) + \
                    ((x[4] + x[5]) + (x[6] + x[7]))
                q = ((x[0] * x[0] + x[1] * x[1]) + (x[2] * x[2] + x[3] * x[3])) + \
                    ((x[4] * x[4] + x[5] * x[5]) + (x[6] * x[6] + x[7] * x[7]))
                for p in perms:
                    s = s + s.at[p].get(mode="promise_in_bounds")
                    q = q + q.at[p].get(mode="promise_in_bounds")
                mean_v = s * (1.0 / HIDDEN)
                var_v = q * (1.0 / HIDDEN) - mean_v * mean_v
                rstd_v = _rsqrt(var_v + EPS)
                for j in range(NVEC):
                    rows_b[b, i, pl.ds(LANES * j, LANES)] = \
                        (x[j] - mean_v) * rstd_v * gs[j] + bs[j]

        gather_desc(0, 0).start()
        gather_desc(1, 1).start()
        gather_desc(0, 0).wait()
        compute_chunk(0, 0)
        wb_desc(0, 0).start()

        pltpu.make_async_copy(
            rows_b.at[0], out_hbm.at[pl.ds(0, CHUNK)], wsems[0]).wait()
        gather_desc(2, 0).start()
        gather_desc(1, 1).wait()
        compute_chunk(1, 1)
        wb_desc(1, 1).start()

        def pair_body(cp, carry):
            for b in range(NBUF):
                c = cp * NBUF + b
                nb = 1 - b
                pltpu.make_async_copy(
                    rows_b.at[nb], out_hbm.at[pl.ds(0, CHUNK)], wsems[nb]
                ).wait()
                gather_desc(c + 1, nb).start()
                gather_desc(c, b).wait()
                compute_chunk(c, b)
                wb_desc(c, b).start()
            return carry

        lax.fori_loop(1, n_chunks // NBUF, pair_body, 0)

        gather_desc(n_chunks, 0).wait()
        wb_desc(n_chunks - 1, 1).wait()

    out = emb_kernel(ids, word_emb, pos_emb, gamma, beta)
    return out.reshape(B, L, HIDDEN)
